# baseline (speedup 1.0000x reference)
import functools

import jax
import jax.numpy as jnp
from jax import lax
from jax.experimental import pallas as pl
from jax.experimental.pallas import tpu as pltpu
from jax.experimental.pallas import tpu_sc as plsc

_VOCAB = 100000
_DIN = 64
_DOUT = 20
_DPAD = 32
_BATCH = 16384
_SEQ = 50
_N = _BATCH * _SEQ
_Q = 128 // _DPAD

_TR = 4000
_IB = 128

_AB = _BATCH // _Q
_UA = 512
_UI = _AB // _UA


def _table_body(e0, e1, e2, e3, w_ref, b_ref, t_ref):
    parts = []
    for e in (e0, e1, e2, e3):
        logits = jnp.dot(e[...], w_ref[...], preferred_element_type=jnp.float32)
        parts.append(jax.nn.sigmoid(logits + b_ref[...]))
    t_ref[...] = jnp.concatenate(parts, axis=1)


def _project_table(emb_r, w, b):
    nblk = _VOCAB // _TR
    pr = _TR // _Q
    espec = lambda q: pl.BlockSpec((pr, _DIN), lambda i, q=q: (q * nblk + i, 0))
    return pl.pallas_call(
        _table_body,
        grid=(nblk,),
        in_specs=[
            espec(0), espec(1), espec(2), espec(3),
            pl.BlockSpec((_DIN, _DPAD), lambda i: (0, 0)),
            pl.BlockSpec((1, _DPAD), lambda i: (0, 0)),
        ],
        out_specs=pl.BlockSpec((pr, 128), lambda i: (i, 0)),
        out_shape=jax.ShapeDtypeStruct((_VOCAB // _Q, 128), jnp.float32),
    )(emb_r, emb_r, emb_r, emb_r, w, b)


@functools.lru_cache(maxsize=1)
def _make_gather():
    info = plsc.get_sparse_core_info()
    nc, ns = info.num_cores, info.num_subcores
    nw = nc * ns
    per_w = _N // nw
    steps = per_w // _IB
    mesh = plsc.VectorSubcoreMesh(core_axis_name="c", subcore_axis_name="s")

    grp = 4
    ngrp = steps // grp

    @functools.partial(
        pl.kernel,
        mesh=mesh,
        out_type=jax.ShapeDtypeStruct((_N, _DPAD), jnp.float32),
        scratch_types=[
            pltpu.VMEM((steps, _IB), jnp.int32),
            pltpu.VMEM((2, grp * _IB, _DPAD), jnp.float32),
            pltpu.SemaphoreType.DMA,
            pltpu.SemaphoreType.DMA,
            pltpu.SemaphoreType.DMA,
            pltpu.SemaphoreType.DMA,
        ],
        compiler_params=pltpu.CompilerParams(use_tc_tiling_on_sc=False),
    )
    def gather(table_hbm, idx_hbm, out_hbm, idx_v, rows_v, gs0, gs1, ws0, ws1):
        gs = (gs0, gs1)
        ws = (ws0, ws1)
        wid = lax.axis_index("s") * nc + lax.axis_index("c")
        pltpu.sync_copy(idx_hbm.at[pl.ds(wid * steps, steps)], idx_v)
        base = wid * per_w
        grows = grp * _IB

        def fire(g, b):
            for r in range(grp):
                pltpu.async_copy(
                    table_hbm.at[idx_v.at[g * grp + r]],
                    rows_v.at[b].at[pl.ds(r * _IB, _IB)],
                    gs[b],
                )

        def drain(sem, b):
            pltpu.make_async_copy(
                out_hbm.at[pl.ds(base, grows)], rows_v.at[b], sem
            ).wait()

        fire(0, 0)

        @pl.loop(0, ngrp // 2)
        def _(p):
            for b in (0, 1):
                g = 2 * p + b
                b2 = 1 - b

                @pl.when(g + 1 < ngrp)
                def _():
                    @pl.when(g >= 1)
                    def _():
                        drain(ws[b2], b2)

                    fire(g + 1, b2)

                drain(gs[b], b)
                pltpu.async_copy(
                    rows_v.at[b], out_hbm.at[pl.ds(base + g * grows, grows)], ws[b]
                )

        drain(ws[0], 0)
        drain(ws[1], 1)

    return gather


def _unpack_body(p_ref, e_ref, o_ref):
    o_ref[...] = lax.dot_general(
        e_ref[0], p_ref[...], (((1,), (2,)), ((), ())),
        preferred_element_type=jnp.float32,
    )


def _unpack(packed3, esel):
    return pl.pallas_call(
        _unpack_body,
        grid=(_UI, _Q),
        in_specs=[
            pl.BlockSpec((_SEQ, _UA, 128), lambda i, q: (0, i, 0)),
            pl.BlockSpec((1, _DOUT, 128), lambda i, q: (q, 0, 0)),
        ],
        out_specs=pl.BlockSpec(
            (_DOUT, _SEQ, _UA), lambda i, q: (0, 0, q * _UI + i)
        ),
        out_shape=jax.ShapeDtypeStruct((_DOUT, _SEQ, _BATCH), jnp.float32),
    )(packed3, esel)


def _selectors():
    j = jnp.arange(128)
    d = jnp.arange(_DOUT)
    e = (j[None, None, :] == (32 * jnp.arange(_Q)[:, None, None] + d[None, :, None]))
    return e.astype(jnp.float32)


def kernel(x, emb, W, b):
    wp = jnp.pad(W, ((0, 0), (0, _DPAD - _DOUT)))
    bp = jnp.pad(b, (0, _DPAD - _DOUT)).reshape(1, _DPAD)
    emb_r = emb.reshape(_VOCAB // _Q, _Q, _DIN).transpose(1, 0, 2).reshape(_VOCAB, _DIN)
    table = _project_table(emb_r, wp, bp)

    idx_perm = (
        x.astype(jnp.int32).reshape(_Q, _AB, _SEQ).transpose(2, 1, 0)
        .reshape(_N // _IB, _IB)
    )

    out_sc = _make_gather()(table.reshape(_VOCAB, _DPAD), idx_perm)
    packed3 = out_sc.reshape(_SEQ, _AB, 128)
    final_t = _unpack(packed3, _selectors())
    return final_t.transpose(2, 1, 0)


# speedup vs baseline: 1.4992x; 1.4992x over previous
import functools

import jax
import jax.numpy as jnp
from jax import lax
from jax.experimental import pallas as pl
from jax.experimental.pallas import tpu as pltpu
from jax.experimental.pallas import tpu_sc as plsc

_VOCAB = 100000
_DIN = 64
_DOUT = 20
_DPAD = 32
_BATCH = 16384
_SEQ = 50
_N = _BATCH * _SEQ
_Q = 128 // _DPAD

_TR = 4000
_IB = 128

_AB = _BATCH // _Q
_UA = 128
_UI = _AB // _UA


def _table_body(e0, e1, e2, e3, w_ref, b_ref, t_ref):
    parts = []
    for e in (e0, e1, e2, e3):
        logits = jnp.dot(e[...], w_ref[...], preferred_element_type=jnp.float32)
        parts.append(jax.nn.sigmoid(logits + b_ref[...]))
    t_ref[...] = jnp.concatenate(parts, axis=1)


def _project_table(emb_r, w, b):
    nblk = _VOCAB // _TR
    pr = _TR // _Q
    espec = lambda q: pl.BlockSpec((pr, _DIN), lambda i, q=q: (q * nblk + i, 0))
    return pl.pallas_call(
        _table_body,
        grid=(nblk,),
        in_specs=[
            espec(0), espec(1), espec(2), espec(3),
            pl.BlockSpec((_DIN, _DPAD), lambda i: (0, 0)),
            pl.BlockSpec((1, _DPAD), lambda i: (0, 0)),
        ],
        out_specs=pl.BlockSpec((pr, 128), lambda i: (i, 0)),
        out_shape=jax.ShapeDtypeStruct((_VOCAB // _Q, 128), jnp.float32),
    )(emb_r, emb_r, emb_r, emb_r, w, b)


@functools.lru_cache(maxsize=1)
def _make_gather():
    info = plsc.get_sparse_core_info()
    nc, ns = info.num_cores, info.num_subcores
    nw = nc * ns
    per_w = _N // nw
    steps = per_w // _IB
    mesh = plsc.VectorSubcoreMesh(core_axis_name="c", subcore_axis_name="s")

    grp = 4
    ngrp = steps // grp

    @functools.partial(
        pl.kernel,
        mesh=mesh,
        out_type=jax.ShapeDtypeStruct((_N, _DPAD), jnp.float32),
        scratch_types=[
            pltpu.VMEM((steps, _IB), jnp.int32),
            pltpu.VMEM((2, grp * _IB, _DPAD), jnp.float32),
            pltpu.SemaphoreType.DMA,
            pltpu.SemaphoreType.DMA,
            pltpu.SemaphoreType.DMA,
            pltpu.SemaphoreType.DMA,
        ],
        compiler_params=pltpu.CompilerParams(use_tc_tiling_on_sc=False),
    )
    def gather(table_hbm, idx_hbm, out_hbm, idx_v, rows_v, gs0, gs1, ws0, ws1):
        gs = (gs0, gs1)
        ws = (ws0, ws1)
        wid = lax.axis_index("s") * nc + lax.axis_index("c")
        pltpu.sync_copy(idx_hbm.at[pl.ds(wid * steps, steps)], idx_v)
        base = wid * per_w
        grows = grp * _IB

        def fire(g, b):
            for r in range(grp):
                pltpu.async_copy(
                    table_hbm.at[idx_v.at[g * grp + r]],
                    rows_v.at[b].at[pl.ds(r * _IB, _IB)],
                    gs[b],
                )

        def drain(sem, b):
            pltpu.make_async_copy(
                out_hbm.at[pl.ds(base, grows)], rows_v.at[b], sem
            ).wait()

        fire(0, 0)

        @pl.loop(0, ngrp // 2)
        def _(p):
            for b in (0, 1):
                g = 2 * p + b
                b2 = 1 - b

                @pl.when(g + 1 < ngrp)
                def _():
                    @pl.when(g >= 1)
                    def _():
                        drain(ws[b2], b2)

                    fire(g + 1, b2)

                drain(gs[b], b)
                pltpu.async_copy(
                    rows_v.at[b], out_hbm.at[pl.ds(base + g * grows, grows)], ws[b]
                )

        drain(ws[0], 0)
        drain(ws[1], 1)

    return gather


def _unpack_body(p_ref, e_ref, il_ref, o_ref):
    parts = [
        lax.dot_general(
            e_ref[q], p_ref[...], (((1,), (2,)), ((), ())),
            preferred_element_type=jnp.float32,
        )
        for q in range(_Q)
    ]
    c = jnp.concatenate(parts, axis=2)
    o_ref[...] = lax.dot_general(
        c, il_ref[...], (((2,), (0,)), ((), ())),
        preferred_element_type=jnp.float32,
    )


def _unpack(packed3, esel, ilv):
    return pl.pallas_call(
        _unpack_body,
        grid=(_UI,),
        in_specs=[
            pl.BlockSpec((_SEQ, _UA, 128), lambda i: (0, i, 0)),
            pl.BlockSpec((_Q, _DOUT, 128), lambda i: (0, 0, 0)),
            pl.BlockSpec((_Q * _UA, _Q * _UA), lambda i: (0, 0)),
        ],
        out_specs=pl.BlockSpec(
            (_DOUT, _SEQ, _Q * _UA), lambda i: (0, 0, i)
        ),
        out_shape=jax.ShapeDtypeStruct((_DOUT, _SEQ, _BATCH), jnp.float32),
    )(packed3, esel, ilv)


def _selectors():
    j = jnp.arange(128)
    d = jnp.arange(_DOUT)
    e = (j[None, None, :] == (32 * jnp.arange(_Q)[:, None, None] + d[None, :, None]))
    return e.astype(jnp.float32)


def _interleaver():
    g = jnp.arange(_Q * _UA)
    c = jnp.arange(_Q * _UA)
    return ((c[None, :] % _Q) * _UA + c[None, :] // _Q == g[:, None]).astype(
        jnp.float32
    )


def kernel(x, emb, W, b):
    wp = jnp.pad(W, ((0, 0), (0, _DPAD - _DOUT)))
    bp = jnp.pad(b, (0, _DPAD - _DOUT)).reshape(1, _DPAD)
    emb_r = emb.reshape(_VOCAB // _Q, _Q, _DIN).transpose(1, 0, 2).reshape(_VOCAB, _DIN)
    table = _project_table(emb_r, wp, bp)

    idx_nat = x.astype(jnp.int32).T.reshape(_N // _IB, _IB)

    out_sc = _make_gather()(table.reshape(_VOCAB, _DPAD), idx_nat)
    packed3 = out_sc.reshape(_SEQ, _AB, 128)
    final_t = _unpack(packed3, _selectors(), _interleaver())
    return final_t.transpose(2, 1, 0)
